# baseline (speedup 1.0000x reference)
import jax
import jax.numpy as jnp
from jax.experimental import pallas as pl
from jax.experimental.pallas import tpu as pltpu

B = 16384
K = 64
CH = 1024
N = B // CH
D = 16
P = 12


def _body(gu_hbm, gi_hbm, xui_hbm, ubuf, vbuf, xbuf, uin, vin, xsem):
    def start_in(c):
        s = c % D
        pltpu.make_async_copy(gu_hbm.at[pl.ds(c * CH, CH), :], ubuf.at[s],
                              uin.at[s]).start(priority=c % 2)
        pltpu.make_async_copy(gi_hbm.at[pl.ds(c * CH, CH), :], vbuf.at[s],
                              vin.at[s]).start(priority=(c + 1) % 2)

    def wait_in(c):
        s = c % D
        pltpu.make_async_copy(gu_hbm.at[pl.ds(c * CH, CH), :], ubuf.at[s],
                              uin.at[s]).wait()
        pltpu.make_async_copy(gi_hbm.at[pl.ds(c * CH, CH), :], vbuf.at[s],
                              vin.at[s]).wait()

    for c in range(P):
        start_in(c)

    for c in range(N):
        s = c % D
        wait_in(c)
        if c + P < N:
            start_in(c + P)
        xbuf[pl.ds(c * CH, CH)] = jnp.sum(ubuf[s] * vbuf[s], axis=1)

    cp = pltpu.make_async_copy(xbuf, xui_hbm, xsem)
    cp.start()
    cp.wait()


def kernel(gu, gi):
    xui = pl.pallas_call(
        _body,
        in_specs=[
            pl.BlockSpec(memory_space=pl.ANY),
            pl.BlockSpec(memory_space=pl.ANY),
        ],
        out_specs=pl.BlockSpec(memory_space=pl.ANY),
        out_shape=jax.ShapeDtypeStruct((B,), gu.dtype),
        scratch_shapes=[
            pltpu.VMEM((D, CH, K), jnp.float32),
            pltpu.VMEM((D, CH, K), jnp.float32),
            pltpu.VMEM((B,), jnp.float32),
            pltpu.SemaphoreType.DMA((D,)),
            pltpu.SemaphoreType.DMA((D,)),
            pltpu.SemaphoreType.DMA,
        ],
    )(gu, gi)
    return (xui, gu, gi)
